# baseline (speedup 1.0000x reference)
import jax
import jax.numpy as jnp
from jax import lax
from jax.experimental import pallas as pl
from jax.experimental.pallas import tpu as pltpu
from jax.experimental.pallas import tpu_sc as plsc

B = 16
H = 8
MAX_SEQ = 4096
S = 16
D = 128

NC = 2
NS = 16
NW = NC * NS
ROWS = B * H * S
RPW = ROWS // NW
BPW = RPW // S

_SK = 0
_SV = RPW
_CK = 2 * RPW
_CV = 3 * RPW


def _body(kv, vv, kc, vc, pos_h, ko, vo, buf, idx, posb, sem_pos, sem):
    wid = lax.axis_index("s") * NC + lax.axis_index("c")
    base = wid * RPW

    st_k = pltpu.async_copy(kv.at[pl.ds(base, RPW)], buf.at[pl.ds(_SK, RPW)],
                            sem)
    st_v = pltpu.async_copy(vv.at[pl.ds(base, RPW)], buf.at[pl.ds(_SV, RPW)],
                            sem)

    pltpu.async_copy(pos_h.at[pl.ds(0, S)], posb, sem_pos).wait()
    pos = posb[...]
    inb = jnp.logical_and(pos >= 0, pos < S)
    pos_safe = jnp.where(inb, pos, 0)
    mask = 0
    for j in range(S):
        pj = pos[j]
        valid = jnp.logical_and(pj >= 0, pj < S)
        bit = jnp.where(valid, lax.shift_left(1, pj), 0)
        mask = lax.bitwise_or(mask, bit)
    allcov = mask == (1 << S) - 1

    @pl.when(allcov)
    def _fast():
        for t in range(BPW):
            bh = wid * BPW + t
            idx[pl.ds(t * S, S)] = bh * S + pos_safe
        st_k.wait()
        st_v.wait()
        sc_k = pltpu.async_copy(buf.at[pl.ds(_SK, RPW)], ko.at[idx], sem)
        sc_v = pltpu.async_copy(buf.at[pl.ds(_SV, RPW)], vo.at[idx], sem)
        sc_k.wait()
        sc_v.wait()

    @pl.when(jnp.logical_not(allcov))
    def _general():
        cps = [st_k, st_v]
        for t in range(BPW):
            bh = wid * BPW + t
            cps.append(pltpu.async_copy(
                kc.at[pl.ds(bh * MAX_SEQ, S)],
                buf.at[pl.ds(_CK + t * S, S)], sem))
            cps.append(pltpu.async_copy(
                vc.at[pl.ds(bh * MAX_SEQ, S)],
                buf.at[pl.ds(_CV + t * S, S)], sem))
        for c in cps:
            c.wait()
        w_k = pltpu.async_copy(buf.at[pl.ds(_CK, RPW)],
                               ko.at[pl.ds(base, RPW)], sem)
        w_v = pltpu.async_copy(buf.at[pl.ds(_CV, RPW)],
                               vo.at[pl.ds(base, RPW)], sem)
        w_k.wait()
        w_v.wait()
        for j in range(S):
            p_j = pos[j]
            ok_j = jnp.logical_and(p_j >= 0, p_j < S)

            @pl.when(ok_j)
            def _row(p_j=p_j, j=j):
                p_c = jnp.clip(p_j, 0, S - 1)
                for t in range(BPW):
                    bh = wid * BPW + t
                    pltpu.sync_copy(buf.at[pl.ds(_SK + t * S + j, 1)],
                                    ko.at[pl.ds(bh * S + p_c, 1)])
                    pltpu.sync_copy(buf.at[pl.ds(_SV + t * S + j, 1)],
                                    vo.at[pl.ds(bh * S + p_c, 1)])


@jax.jit
def _cache_update(kv, vv, kc, vc, cache_pos):
    mesh = plsc.VectorSubcoreMesh(core_axis_name="c", subcore_axis_name="s",
                                  num_cores=NC, num_subcores=NS)
    out = jax.ShapeDtypeStruct((ROWS, D), jnp.float32)
    ko, vo = pl.kernel(
        _body,
        out_type=(out, out),
        mesh=mesh,
        scratch_types=[
            pltpu.VMEM((4 * RPW, D), jnp.float32),
            pltpu.VMEM((RPW,), jnp.int32),
            pltpu.VMEM((S,), jnp.int32),
            pltpu.SemaphoreType.DMA,
            pltpu.SemaphoreType.DMA,
        ],
    )(kv, vv, kc, vc, cache_pos)
    return ko, vo


def kernel(k_val, v_val, k_cache, v_cache, cache_pos):
    kv = k_val.reshape(ROWS, D)
    vv = v_val.reshape(ROWS, D)
    kc = k_cache.reshape(B * H * MAX_SEQ, D)
    vc = v_cache.reshape(B * H * MAX_SEQ, D)
    ko, vo = _cache_update(kv, vv, kc, vc, cache_pos)
    k_ret = ko.reshape(B, H, S, D)
    v_ret = vo.reshape(B, H, S, D)
    return (k_ret, v_ret)


# speedup vs baseline: 1.0062x; 1.0062x over previous
import jax
import jax.numpy as jnp
from jax import lax
from jax.experimental import pallas as pl
from jax.experimental.pallas import tpu as pltpu
from jax.experimental.pallas import tpu_sc as plsc

B = 16
H = 8
MAX_SEQ = 4096
S = 16
D = 128

NC = 2
NS = 16
NW = NC * NS
ROWS = B * H * S
RPW = ROWS // NW
BPW = RPW // S

_SK = 0
_SV = RPW
_CK = 2 * RPW
_CV = 3 * RPW


def _body(kv, vv, kc, vc, pos_h, ko, vo, buf, idx, posb, sem_pos, sem_v, sem):
    wid = lax.axis_index("s") * NC + lax.axis_index("c")
    base = wid * RPW

    cp_pos = pltpu.async_copy(pos_h.at[pl.ds(0, S)], posb, sem_pos)
    st_k = pltpu.async_copy(kv.at[pl.ds(base, RPW)], buf.at[pl.ds(_SK, RPW)],
                            sem)
    st_v = pltpu.async_copy(vv.at[pl.ds(base, RPW)], buf.at[pl.ds(_SV, RPW)],
                            sem_v)

    cp_pos.wait()
    pos = posb[...]
    inb = jnp.logical_and(pos >= 0, pos < S)
    pos_safe = jnp.where(inb, pos, 0)
    mask = 0
    for j in range(S):
        pj = pos[j]
        valid = jnp.logical_and(pj >= 0, pj < S)
        bit = jnp.where(valid, lax.shift_left(1, pj), 0)
        mask = lax.bitwise_or(mask, bit)
    allcov = mask == (1 << S) - 1

    @pl.when(allcov)
    def _fast():
        for t in range(BPW):
            bh = wid * BPW + t
            idx[pl.ds(t * S, S)] = bh * S + pos_safe
        st_k.wait()
        sc_k = pltpu.async_copy(buf.at[pl.ds(_SK, RPW)], ko.at[idx], sem)
        st_v.wait()
        sc_v = pltpu.async_copy(buf.at[pl.ds(_SV, RPW)], vo.at[idx], sem)
        sc_k.wait()
        sc_v.wait()

    @pl.when(jnp.logical_not(allcov))
    def _general():
        cps = [st_k, st_v]
        for t in range(BPW):
            bh = wid * BPW + t
            cps.append(pltpu.async_copy(
                kc.at[pl.ds(bh * MAX_SEQ, S)],
                buf.at[pl.ds(_CK + t * S, S)], sem))
            cps.append(pltpu.async_copy(
                vc.at[pl.ds(bh * MAX_SEQ, S)],
                buf.at[pl.ds(_CV + t * S, S)], sem))
        for c in cps:
            c.wait()
        w_k = pltpu.async_copy(buf.at[pl.ds(_CK, RPW)],
                               ko.at[pl.ds(base, RPW)], sem)
        w_v = pltpu.async_copy(buf.at[pl.ds(_CV, RPW)],
                               vo.at[pl.ds(base, RPW)], sem)
        w_k.wait()
        w_v.wait()
        for j in range(S):
            p_j = pos[j]
            ok_j = jnp.logical_and(p_j >= 0, p_j < S)

            @pl.when(ok_j)
            def _row(p_j=p_j, j=j):
                p_c = jnp.clip(p_j, 0, S - 1)
                for t in range(BPW):
                    bh = wid * BPW + t
                    pltpu.sync_copy(buf.at[pl.ds(_SK + t * S + j, 1)],
                                    ko.at[pl.ds(bh * S + p_c, 1)])
                    pltpu.sync_copy(buf.at[pl.ds(_SV + t * S + j, 1)],
                                    vo.at[pl.ds(bh * S + p_c, 1)])


@jax.jit
def _cache_update(kv, vv, kc, vc, cache_pos):
    mesh = plsc.VectorSubcoreMesh(core_axis_name="c", subcore_axis_name="s",
                                  num_cores=NC, num_subcores=NS)
    out = jax.ShapeDtypeStruct((ROWS, D), jnp.float32)
    ko, vo = pl.kernel(
        _body,
        out_type=(out, out),
        mesh=mesh,
        scratch_types=[
            pltpu.VMEM((4 * RPW, D), jnp.float32),
            pltpu.VMEM((RPW,), jnp.int32),
            pltpu.VMEM((S,), jnp.int32),
            pltpu.SemaphoreType.DMA,
            pltpu.SemaphoreType.DMA,
            pltpu.SemaphoreType.DMA,
        ],
    )(kv, vv, kc, vc, cache_pos)
    return ko, vo


def kernel(k_val, v_val, k_cache, v_cache, cache_pos):
    kv = k_val.reshape(ROWS, D)
    vv = v_val.reshape(ROWS, D)
    kc = k_cache.reshape(B * H * MAX_SEQ, D)
    vc = v_cache.reshape(B * H * MAX_SEQ, D)
    ko, vo = _cache_update(kv, vv, kc, vc, cache_pos)
    k_ret = ko.reshape(B, H, S, D)
    v_ret = vo.reshape(B, H, S, D)
    return (k_ret, v_ret)
